# Initial kernel scaffold; baseline (speedup 1.0000x reference)
import jax
import jax.numpy as jnp
from jax.experimental import pallas as pl


def kernel(x):
    raise NotImplementedError("write your pallas kernel here")


# speedup vs baseline: 18.8608x; 18.8608x over previous
import jax
import jax.numpy as jnp
from jax.experimental import pallas as pl

_P = 4
_H = 96
_W = 96
_C = 96
_B = 32


def _body(x_ref, o_ref):
    n = pl.program_id(0)
    m = jax.lax.rem(n, _P * _P)
    py = jax.lax.div(m, _P)
    px = jax.lax.rem(m, _P)
    is_top = py == 0
    is_bot = py == _P - 1
    is_left = px == 0
    is_right = px == _P - 1

    xb = x_ref[0]
    third = jnp.float32(1.0 / 3.0)

    o_ref[0, :, 1:_H + 1, 1:_W + 1] = xb

    r_first = xb[:, 0, :]
    r_last = xb[:, _H - 1, :]

    def mean3_right0(r):
        z = jnp.zeros((r.shape[0], 2), r.dtype)
        rp = jnp.concatenate([r, z], axis=1)
        return (rp[:, 0:_W] + rp[:, 1:_W + 1] + rp[:, 2:_W + 2]) * third

    top_mid = mean3_right0(r_first)
    bot_mid = mean3_right0(r_last)

    row_top = jnp.concatenate([r_first[:, :1], top_mid, r_first[:, _W - 1:]], axis=1)
    row_bot = jnp.concatenate([r_last[:, :1], bot_mid, r_last[:, _W - 1:]], axis=1)

    cidx = jax.lax.broadcasted_iota(jnp.int32, (1, _W + 2), 1)
    col_zero = (is_left & (cidx == 0)) | (is_right & (cidx == _W + 1))
    row_top = jnp.where(col_zero | is_top, 0.0, row_top)
    row_bot = jnp.where(col_zero | is_bot, 0.0, row_bot)
    o_ref[0, :, 0:1, :] = row_top[:, None, :]
    o_ref[0, :, _H + 1:_H + 2, :] = row_bot[:, None, :]

    left = (xb[:, :, 0] + xb[:, :, 1] + xb[:, :, 2]) * third
    right = (xb[:, :, _W - 3] + xb[:, :, _W - 2] + xb[:, :, _W - 1]) * third
    left = jnp.where(is_left, 0.0, left)
    right = jnp.where(is_right, 0.0, right)
    o_ref[0, :, 1:_H + 1, 0:1] = left[:, :, None]
    o_ref[0, :, 1:_H + 1, _W + 1:_W + 2] = right[:, :, None]


def kernel(x):
    return pl.pallas_call(
        _body,
        grid=(_B,),
        in_specs=[pl.BlockSpec((1, _C, _H, _W), lambda n: (n, 0, 0, 0))],
        out_specs=pl.BlockSpec((1, _C, _H + 2, _W + 2), lambda n: (n, 0, 0, 0)),
        out_shape=jax.ShapeDtypeStruct((_B, _C, _H + 2, _W + 2), jnp.float32),
    )(x)
